# Initial kernel scaffold; baseline (speedup 1.0000x reference)
import jax
import jax.numpy as jnp
from jax.experimental import pallas as pl


def kernel(cluster_ids, embedding_weight):
    raise NotImplementedError("write your pallas kernel here")


# speedup vs baseline: 1.7461x; 1.7461x over previous
import functools

import jax
import jax.numpy as jnp
from jax import lax
from jax.experimental import pallas as pl
from jax.experimental.pallas import tpu as pltpu
from jax.experimental.pallas import tpu_sc as plsc

N_CLUSTERS = 100
EMBED_DIM = 64
BATCH = 16384

_NC = 2
_NS = 16
_NW = _NC * _NS
_B_PER_W = BATCH // _NW
_CHUNK = 128
_NCHUNK = _B_PER_W // _CHUNK


def _sc_embedding_gather(ids3, table):
    mesh = plsc.VectorSubcoreMesh(core_axis_name="c", subcore_axis_name="s")

    @functools.partial(
        pl.kernel,
        mesh=mesh,
        out_type=jax.ShapeDtypeStruct((BATCH, EMBED_DIM), jnp.float32),
        scratch_types=[
            pltpu.VMEM((_NCHUNK, _CHUNK), jnp.int32),
            pltpu.VMEM((_B_PER_W, EMBED_DIM), jnp.float32),
            pltpu.SemaphoreType.DMA,
        ],
        compiler_params=pltpu.CompilerParams(use_tc_tiling_on_sc=False),
    )
    def k(ids_hbm, table_hbm, out_hbm, idx_v, rows_v, sem):
        wid = lax.axis_index("s") * _NC + lax.axis_index("c")
        base = wid * _B_PER_W
        pltpu.sync_copy(ids_hbm.at[wid], idx_v)
        copies = []
        for j in range(_NCHUNK):
            copies.append(
                pltpu.async_copy(
                    table_hbm.at[idx_v.at[j]],
                    rows_v.at[pl.ds(j * _CHUNK, _CHUNK)],
                    sem,
                )
            )
        for c in copies:
            c.wait()
        pltpu.sync_copy(rows_v, out_hbm.at[pl.ds(base, _B_PER_W)])

    return k(ids3, table)


def kernel(cluster_ids, embedding_weight):
    ids3 = cluster_ids.astype(jnp.int32).reshape(_NW, _NCHUNK, _CHUNK)
    return _sc_embedding_gather(ids3, embedding_weight)


# speedup vs baseline: 2.5042x; 1.4341x over previous
import functools

import jax
import jax.numpy as jnp
from jax import lax
from jax.experimental import pallas as pl
from jax.experimental.pallas import tpu as pltpu
from jax.experimental.pallas import tpu_sc as plsc

N_CLUSTERS = 100
EMBED_DIM = 64
BATCH = 16384

_NC = 2
_NS = 16
_NW = _NC * _NS
_B_PER_W = BATCH // _NW
_CHUNK = 128
_NCHUNK = _B_PER_W // _CHUNK


def _sc_embedding_gather(ids, table):
    mesh = plsc.VectorSubcoreMesh(core_axis_name="c", subcore_axis_name="s")

    @functools.partial(
        pl.kernel,
        mesh=mesh,
        out_type=jax.ShapeDtypeStruct((BATCH, EMBED_DIM), jnp.float32),
        scratch_types=[
            pltpu.VMEM((_B_PER_W,), jnp.int32),
            pltpu.VMEM((_B_PER_W, EMBED_DIM), jnp.float32),
            pltpu.VMEM_SHARED((N_CLUSTERS, EMBED_DIM), jnp.float32),
            pltpu.SemaphoreType.DMA,
            pltpu.SemaphoreType.DMA,
        ],
    )
    def k(ids_hbm, table_hbm, out_hbm, idx_v, rows_v, table_sh, sem_g, sem_w):
        sid = lax.axis_index("s")
        wid = sid * _NC + lax.axis_index("c")
        base = wid * _B_PER_W

        @pl.when(sid == 0)
        def _stage_table():
            pltpu.sync_copy(table_hbm, table_sh)

        ids_cp = pltpu.async_copy(ids_hbm.at[pl.ds(base, _B_PER_W)], idx_v, sem_g)
        plsc.subcore_barrier()
        ids_cp.wait()

        gathers = []
        for j in range(_NCHUNK):
            gathers.append(
                pltpu.async_copy(
                    table_sh.at[idx_v.at[pl.ds(j * _CHUNK, _CHUNK)]],
                    rows_v.at[pl.ds(j * _CHUNK, _CHUNK)],
                    sem_g,
                )
            )
        writes = []
        for j in range(_NCHUNK):
            gathers[j].wait()
            writes.append(
                pltpu.async_copy(
                    rows_v.at[pl.ds(j * _CHUNK, _CHUNK)],
                    out_hbm.at[pl.ds(base + j * _CHUNK, _CHUNK)],
                    sem_w,
                )
            )
        for w in writes:
            w.wait()

    return k(ids, table)


def kernel(cluster_ids, embedding_weight):
    ids = cluster_ids.astype(jnp.int32)
    return _sc_embedding_gather(ids, embedding_weight)
